# Initial kernel scaffold; baseline (speedup 1.0000x reference)
import jax
import jax.numpy as jnp
from jax.experimental import pallas as pl


def kernel(h, coords, edge_index, We1, be1, We2, be2, We3, be3, Wn1, bn1, Wn2, bn2, Wn3, bn3, Wc1, bc1, Wc2, gamma, beta):
    raise NotImplementedError("write your pallas kernel here")


# speedup vs baseline: 2.4363x; 2.4363x over previous
import functools

import jax
import jax.numpy as jnp
from jax import lax
from jax.experimental import pallas as pl
from jax.experimental.pallas import tpu as pltpu
from jax.experimental.pallas import tpu_sc as plsc

N = 10000
E = 320000
D = 128

NC = 2
NS = 16
NW = NC * NS

GB = 50
RPC = 8
CHUNK = GB * RPC
IDX_ROWS = E // GB
CPW = IDX_ROWS // RPC // NW

CH2 = 128
IDX2_ROWS = E // 16
NCHUNK2 = E // CH2
CNT2_BASE = NCHUNK2 // NW
CNT2_REM = NCHUNK2 % NW

CR = 1280
AGG_STRIPE = 632
AGG_LAST = N - 15 * AGG_STRIPE
CACC_STRIPE = CR // NS

_mesh = plsc.VectorSubcoreMesh(core_axis_name="c", subcore_axis_name="s")


def _silu(x):
    return x * jax.nn.sigmoid(x)


@functools.partial(
    pl.kernel,
    out_type=[
        jax.ShapeDtypeStruct((E, D), jnp.float32),
        jax.ShapeDtypeStruct((E, D), jnp.float32),
    ],
    mesh=_mesh,
    scratch_types=[
        pltpu.VMEM((RPC, GB), jnp.int32),
        pltpu.VMEM((RPC, GB), jnp.int32),
        pltpu.VMEM((CHUNK, D), jnp.float32),
        pltpu.VMEM((CHUNK, D), jnp.float32),
        pltpu.SemaphoreType.DMA,
    ],
)
def _gather_kernel(h_hbm, row2_hbm, col2_hbm, hrow_hbm, hcol_hbm,
                   idxr_v, idxc_v, hrow_v, hcol_v, sem):
    wid = lax.axis_index("s") * NC + lax.axis_index("c")

    def step(t, carry):
        c = wid * CPW + t
        r0 = c * RPC
        pltpu.sync_copy(row2_hbm.at[pl.ds(r0, RPC)], idxr_v)
        pltpu.sync_copy(col2_hbm.at[pl.ds(r0, RPC)], idxc_v)
        cps = []
        for j in range(RPC):
            dst = pl.ds(j * GB, GB)
            cps.append(pltpu.async_copy(h_hbm.at[idxr_v.at[j]], hrow_v.at[dst], sem))
            cps.append(pltpu.async_copy(h_hbm.at[idxc_v.at[j]], hcol_v.at[dst], sem))
        for cp in cps:
            cp.wait()
        e0 = c * CHUNK
        pltpu.sync_copy(hrow_v, hrow_hbm.at[pl.ds(e0, CHUNK)])
        pltpu.sync_copy(hcol_v, hcol_hbm.at[pl.ds(e0, CHUNK)])
        return carry

    lax.fori_loop(0, CPW, step, 0)


@functools.partial(
    pl.kernel,
    out_type=jax.ShapeDtypeStruct((E * 16,), jnp.float32),
    mesh=_mesh,
    compiler_params=pltpu.CompilerParams(needs_layout_passes=False),
    scratch_types=[
        pltpu.VMEM((N,), jnp.float32),
        pltpu.VMEM((N,), jnp.float32),
        pltpu.VMEM((N,), jnp.float32),
        pltpu.VMEM((8, 16), jnp.int32),
        pltpu.VMEM((8, 16), jnp.int32),
        pltpu.VMEM((CH2 * 16,), jnp.float32),
    ],
)
def _rel_kernel(cx_hbm, cy_hbm, cz_hbm, row16_hbm, col16_hbm, rel_hbm,
                cx_v, cy_v, cz_v, idxr_v, idxc_v, rel_v):
    wid = lax.axis_index("s") * NC + lax.axis_index("c")
    pltpu.sync_copy(cx_hbm, cx_v)
    pltpu.sync_copy(cy_hbm, cy_v)
    pltpu.sync_copy(cz_hbm, cz_v)

    zero16 = jnp.zeros((16,), jnp.float32)

    def zstep(i, carry):
        rel_v[pl.ds(i * 16, 16)] = zero16
        return carry

    lax.fori_loop(0, CH2, zstep, 0)

    lanes = lax.iota(jnp.int32, 16)
    cnt = CNT2_BASE + jnp.where(wid < CNT2_REM, 1, 0)

    def step(t, carry):
        c = wid + t * NW
        rr = c * 8
        pltpu.sync_copy(row16_hbm.at[pl.ds(rr, 8)], idxr_v)
        pltpu.sync_copy(col16_hbm.at[pl.ds(rr, 8)], idxc_v)
        for j in range(8):
            ri = idxr_v.at[j][...]
            ci = idxc_v.at[j][...]
            relx = plsc.load_gather(cx_v, [ri]) - plsc.load_gather(cx_v, [ci])
            rely = plsc.load_gather(cy_v, [ri]) - plsc.load_gather(cy_v, [ci])
            relz = plsc.load_gather(cz_v, [ri]) - plsc.load_gather(cz_v, [ci])
            base = lanes * 16 + (j * 16 * 16)
            plsc.store_scatter(rel_v, [base], relx)
            plsc.store_scatter(rel_v, [base + 1], rely)
            plsc.store_scatter(rel_v, [base + 2], relz)
        pltpu.sync_copy(rel_v, rel_hbm.at[pl.ds(c * (CH2 * 16), CH2 * 16)])
        return carry

    lax.fori_loop(0, cnt, step, 0)


BE = 1000


def _edge_body(hrow, hcol, relc, rowi,
               We1a, We1b, we1d, be1, We2, be2, We3, be3, Wc1, bc1, wc2t,
               msg_o, cww_o):
    rel = relc[...]
    dist2 = jnp.sum(rel * rel, axis=1, keepdims=True)
    m1 = _silu(jnp.dot(hrow[...], We1a[...], preferred_element_type=jnp.float32)
               + jnp.dot(hcol[...], We1b[...], preferred_element_type=jnp.float32)
               + dist2 * we1d[...] + be1[...])
    m2 = _silu(jnp.dot(m1, We2[...], preferred_element_type=jnp.float32) + be2[...])
    msg = jnp.dot(m2, We3[...], preferred_element_type=jnp.float32) + be3[...]
    t = _silu(jnp.dot(msg, Wc1[...], preferred_element_type=jnp.float32) + bc1[...])
    cw = jnp.sum(t * wc2t[...], axis=1, keepdims=True)
    cwrel = cw * rel
    m8 = rowi[...] & 7
    lane_grp = lax.broadcasted_iota(jnp.int32, (BE, D), 1) // 16
    tiled = jnp.concatenate([cwrel] * 8, axis=1)
    msg_o[...] = msg
    cww_o[...] = jnp.where(lane_grp == m8, tiled, 0.0)


def _edge_call(hrow, hcol, relc, rowi, We1a, We1b, we1d, be1, We2, be2,
               We3, be3, Wc1, bc1, wc2t):
    grid = (E // BE,)
    def eb(shape):
        return pl.BlockSpec(shape, lambda i: (i, 0))
    def wb(shape):
        return pl.BlockSpec(shape, lambda i: (0, 0))
    return pl.pallas_call(
        _edge_body,
        grid=grid,
        in_specs=[
            eb((BE, D)), eb((BE, D)), eb((BE, 16)), eb((BE, 1)),
            wb((D, 2 * D)), wb((D, 2 * D)), wb((1, 2 * D)), wb((1, 2 * D)),
            wb((2 * D, D)), wb((1, D)), wb((D, D)), wb((1, D)),
            wb((D, D)), wb((1, D)), wb((1, D)),
        ],
        out_specs=[eb((BE, D)), eb((BE, D))],
        out_shape=[
            jax.ShapeDtypeStruct((E, D), jnp.float32),
            jax.ShapeDtypeStruct((E, D), jnp.float32),
        ],
    )(hrow, hcol, relc, rowi, We1a, We1b, we1d, be1, We2, be2, We3, be3,
      Wc1, bc1, wc2t)


@functools.partial(
    pl.kernel,
    out_type=[
        jax.ShapeDtypeStruct((NC, N, D), jnp.float32),
        jax.ShapeDtypeStruct((NC, CR, D), jnp.float32),
    ],
    mesh=_mesh,
    scratch_types=[
        pltpu.VMEM((8, 16), jnp.int32),
        pltpu.VMEM((8, 16), jnp.int32),
        pltpu.VMEM((CH2, D), jnp.float32),
        pltpu.VMEM((CH2, D), jnp.float32),
        pltpu.VMEM_SHARED((N, D), jnp.float32),
        pltpu.VMEM_SHARED((CR, D), jnp.float32),
        pltpu.SemaphoreType.DMA,
    ],
)
def _scatter_kernel(msg_hbm, cww_hbm, row16_hbm, row8_hbm, zh_hbm,
                    agg_hbm, cacc_hbm,
                    idxr_v, idx8_v, msg_v, cww_v, sh_agg, sh_cacc, sem):
    cid = lax.axis_index("c")
    sid = lax.axis_index("s")
    wid = sid * NC + cid
    a0 = sid * AGG_STRIPE
    asz = jnp.where(sid == NS - 1, AGG_LAST, AGG_STRIPE)
    c0 = sid * CACC_STRIPE
    pltpu.sync_copy(zh_hbm.at[pl.ds(0, asz)], sh_agg.at[pl.ds(a0, asz)])
    pltpu.sync_copy(zh_hbm.at[pl.ds(0, CACC_STRIPE)], sh_cacc.at[pl.ds(c0, CACC_STRIPE)])
    plsc.subcore_barrier()

    cnt = CNT2_BASE + jnp.where(wid < CNT2_REM, 1, 0)

    def step(t, carry):
        c = wid + t * NW
        rr = c * 8
        e0 = c * CH2
        pltpu.sync_copy(row16_hbm.at[pl.ds(rr, 8)], idxr_v)
        pltpu.sync_copy(row8_hbm.at[pl.ds(rr, 8)], idx8_v)
        pltpu.sync_copy(msg_hbm.at[pl.ds(e0, CH2)], msg_v)
        pltpu.sync_copy(cww_hbm.at[pl.ds(e0, CH2)], cww_v)
        for j in range(8):
            src = pl.ds(j * 16, 16)
            pltpu.sync_copy(msg_v.at[src], sh_agg.at[idxr_v.at[j]], add=True)
            pltpu.sync_copy(cww_v.at[src], sh_cacc.at[idx8_v.at[j]], add=True)
        return carry

    lax.fori_loop(0, cnt, step, 0)
    plsc.subcore_barrier()
    pltpu.sync_copy(sh_agg.at[pl.ds(a0, asz)], agg_hbm.at[cid, pl.ds(a0, asz)])
    pltpu.sync_copy(sh_cacc.at[pl.ds(c0, CACC_STRIPE)],
                    cacc_hbm.at[cid, pl.ds(c0, CACC_STRIPE)])


BN = 1000


def _node_body(h, agg2, c16, cacc2, Wn1a, Wn1b, bn1, Wn2, bn2, Wn3, bn3,
               gamma, beta, hnew_o, c16_o):
    agg = agg2[0] + agg2[1]
    u = _silu(jnp.dot(h[...], Wn1a[...], preferred_element_type=jnp.float32)
              + jnp.dot(agg, Wn1b[...], preferred_element_type=jnp.float32)
              + bn1[...])
    u2 = _silu(jnp.dot(u, Wn2[...], preferred_element_type=jnp.float32) + bn2[...])
    x = h[...] + jnp.dot(u2, Wn3[...], preferred_element_type=jnp.float32) + bn3[...]
    mu = jnp.mean(x, axis=1, keepdims=True)
    xc = x - mu
    var = jnp.mean(xc * xc, axis=1, keepdims=True)
    hnew_o[...] = xc * lax.rsqrt(var + 1e-5) * gamma[...] + beta[...]
    c16_o[...] = c16[...] + cacc2[0] + cacc2[1]


def _node_call(h, agg2, c16, cacc2, Wn1a, Wn1b, bn1, Wn2, bn2, Wn3, bn3,
               gamma, beta):
    grid = (N // BN,)
    def nb(shape):
        return pl.BlockSpec(shape, lambda i: (i, 0))
    def pb(shape):
        return pl.BlockSpec(shape, lambda i: (0, i, 0))
    def wb(shape):
        return pl.BlockSpec(shape, lambda i: tuple(0 for _ in shape))
    return pl.pallas_call(
        _node_body,
        grid=grid,
        in_specs=[
            nb((BN, D)), pb((NC, BN, D)), nb((BN, 16)), pb((NC, BN, 16)),
            wb((D, 2 * D)), wb((D, 2 * D)), wb((1, 2 * D)),
            wb((2 * D, D)), wb((1, D)), wb((D, D)), wb((1, D)),
            wb((1, D)), wb((1, D)),
        ],
        out_specs=[nb((BN, D)), nb((BN, 16))],
        out_shape=[
            jax.ShapeDtypeStruct((N, D), jnp.float32),
            jax.ShapeDtypeStruct((N, 16), jnp.float32),
        ],
    )(h, agg2, c16, cacc2, Wn1a, Wn1b, bn1, Wn2, bn2, Wn3, bn3, gamma, beta)


def kernel(h, coords, edge_index, We1, be1, We2, be2, We3, be3,
           Wn1, bn1, Wn2, bn2, Wn3, bn3, Wc1, bc1, Wc2, gamma, beta):
    row = edge_index[0]
    col = edge_index[1]
    row2 = row.reshape(IDX_ROWS, GB)
    col2 = col.reshape(IDX_ROWS, GB)
    row16 = row.reshape(IDX2_ROWS, 16)
    col16 = col.reshape(IDX2_ROWS, 16)
    cx = coords[:, 0]
    cy = coords[:, 1]
    cz = coords[:, 2]
    c16 = jnp.zeros((N, 16), jnp.float32).at[:, :3].set(coords)

    hrow, hcol = _gather_kernel(h, row2, col2)
    relflat = _rel_kernel(cx, cy, cz, row16, col16)
    relc = relflat.reshape(E, 16)

    msg, cww = _edge_call(
        hrow, hcol, relc, row.reshape(E, 1),
        We1[:D], We1[D:2 * D], We1[2 * D:], be1.reshape(1, 2 * D),
        We2, be2.reshape(1, D), We3, be3.reshape(1, D),
        Wc1, bc1.reshape(1, D), Wc2.reshape(1, D))

    zh = jnp.zeros((AGG_STRIPE, D), jnp.float32)
    row8 = (row >> 3).reshape(IDX2_ROWS, 16)
    agg2, cacc2w = _scatter_kernel(msg, cww, row16, row8, zh)
    cacc2 = cacc2w.reshape(NC, CR * 8, 16)[:, :N]

    h_new, c16_o = _node_call(
        h, agg2, c16, cacc2,
        Wn1[:D], Wn1[D:], bn1.reshape(1, 2 * D),
        Wn2, bn2.reshape(1, D), Wn3, bn3.reshape(1, D),
        gamma.reshape(1, D), beta.reshape(1, D))

    return h_new, c16_o[:, :3]


# speedup vs baseline: 3.0215x; 1.2402x over previous
import functools

import jax
import jax.numpy as jnp
from jax import lax
from jax.experimental import pallas as pl
from jax.experimental.pallas import tpu as pltpu
from jax.experimental.pallas import tpu_sc as plsc

N = 10000
E = 320000
D = 128

NC = 2
NS = 16
NW = NC * NS

GB = 50
RPC = 8
CHUNK = GB * RPC
IDX_ROWS = E // GB
CPW = IDX_ROWS // RPC // NW

CH2 = 128
IDX2_ROWS = E // 16
NCHUNK2 = E // CH2
CNT2_BASE = NCHUNK2 // NW
CNT2_REM = NCHUNK2 % NW

CR = 1280
AGG_STRIPE = 632
AGG_LAST = N - 15 * AGG_STRIPE
CACC_STRIPE = CR // NS

_mesh = plsc.VectorSubcoreMesh(core_axis_name="c", subcore_axis_name="s")


def _silu(x):
    return x / (1.0 + jnp.exp(-x))


@functools.partial(
    pl.kernel,
    out_type=[
        jax.ShapeDtypeStruct((E, D), jnp.float32),
        jax.ShapeDtypeStruct((E, D), jnp.float32),
    ],
    mesh=_mesh,
    scratch_types=[
        pltpu.VMEM((RPC, GB), jnp.int32),
        pltpu.VMEM((RPC, GB), jnp.int32),
        pltpu.VMEM((CHUNK, D), jnp.float32),
        pltpu.VMEM((CHUNK, D), jnp.float32),
        pltpu.SemaphoreType.DMA,
    ],
)
def _gather_kernel(h_hbm, row2_hbm, col2_hbm, hrow_hbm, hcol_hbm,
                   idxr_v, idxc_v, hrow_v, hcol_v, sem):
    wid = lax.axis_index("s") * NC + lax.axis_index("c")

    def step(t, carry):
        c = wid * CPW + t
        r0 = c * RPC
        pltpu.sync_copy(row2_hbm.at[pl.ds(r0, RPC)], idxr_v)
        pltpu.sync_copy(col2_hbm.at[pl.ds(r0, RPC)], idxc_v)
        cps = []
        for j in range(RPC):
            dst = pl.ds(j * GB, GB)
            cps.append(pltpu.async_copy(h_hbm.at[idxr_v.at[j]], hrow_v.at[dst], sem))
            cps.append(pltpu.async_copy(h_hbm.at[idxc_v.at[j]], hcol_v.at[dst], sem))
        for cp in cps:
            cp.wait()
        e0 = c * CHUNK
        pltpu.sync_copy(hrow_v, hrow_hbm.at[pl.ds(e0, CHUNK)])
        pltpu.sync_copy(hcol_v, hcol_hbm.at[pl.ds(e0, CHUNK)])
        return carry

    lax.fori_loop(0, CPW, step, 0)


@functools.partial(
    pl.kernel,
    out_type=jax.ShapeDtypeStruct((E * 16,), jnp.float32),
    mesh=_mesh,
    compiler_params=pltpu.CompilerParams(needs_layout_passes=False),
    scratch_types=[
        pltpu.VMEM((N,), jnp.float32),
        pltpu.VMEM((N,), jnp.float32),
        pltpu.VMEM((N,), jnp.float32),
        pltpu.VMEM((8, 16), jnp.int32),
        pltpu.VMEM((8, 16), jnp.int32),
        pltpu.VMEM((CH2 * 16,), jnp.float32),
    ],
)
def _rel_kernel(cx_hbm, cy_hbm, cz_hbm, row16_hbm, col16_hbm, rel_hbm,
                cx_v, cy_v, cz_v, idxr_v, idxc_v, rel_v):
    wid = lax.axis_index("s") * NC + lax.axis_index("c")
    pltpu.sync_copy(cx_hbm, cx_v)
    pltpu.sync_copy(cy_hbm, cy_v)
    pltpu.sync_copy(cz_hbm, cz_v)

    zero16 = jnp.zeros((16,), jnp.float32)

    def zstep(i, carry):
        rel_v[pl.ds(i * 16, 16)] = zero16
        return carry

    lax.fori_loop(0, CH2, zstep, 0)

    lanes = lax.iota(jnp.int32, 16)
    cnt = CNT2_BASE + jnp.where(wid < CNT2_REM, 1, 0)

    def step(t, carry):
        c = wid + t * NW
        rr = c * 8
        pltpu.sync_copy(row16_hbm.at[pl.ds(rr, 8)], idxr_v)
        pltpu.sync_copy(col16_hbm.at[pl.ds(rr, 8)], idxc_v)
        for j in range(8):
            ri = idxr_v.at[j][...]
            ci = idxc_v.at[j][...]
            relx = plsc.load_gather(cx_v, [ri]) - plsc.load_gather(cx_v, [ci])
            rely = plsc.load_gather(cy_v, [ri]) - plsc.load_gather(cy_v, [ci])
            relz = plsc.load_gather(cz_v, [ri]) - plsc.load_gather(cz_v, [ci])
            base = lanes * 16 + (j * 16 * 16)
            plsc.store_scatter(rel_v, [base], relx)
            plsc.store_scatter(rel_v, [base + 1], rely)
            plsc.store_scatter(rel_v, [base + 2], relz)
        pltpu.sync_copy(rel_v, rel_hbm.at[pl.ds(c * (CH2 * 16), CH2 * 16)])
        return carry

    lax.fori_loop(0, cnt, step, 0)


BE = 1000


def _edge_body(hrow, hcol, relc,
               We1a, We1b, we1d, be1, We2, be2, We3, be3, Wc1, bc1, wc2t,
               msg_o, cw_o):
    rel = relc[...]
    dist2 = jnp.sum(rel * rel, axis=1, keepdims=True)
    m1 = _silu(jnp.dot(hrow[...], We1a[...], preferred_element_type=jnp.float32)
               + jnp.dot(hcol[...], We1b[...], preferred_element_type=jnp.float32)
               + dist2 * we1d[...] + be1[...])
    m2 = _silu(jnp.dot(m1, We2[...], preferred_element_type=jnp.float32) + be2[...])
    msg = jnp.dot(m2, We3[...], preferred_element_type=jnp.float32) + be3[...]
    t = _silu(jnp.dot(msg, Wc1[...], preferred_element_type=jnp.float32) + bc1[...])
    msg_o[...] = msg
    cw_o[...] = jnp.sum(t * wc2t[...], axis=1, keepdims=True)


def _edge_call(hrow, hcol, relc, We1a, We1b, we1d, be1, We2, be2,
               We3, be3, Wc1, bc1, wc2t):
    grid = (E // BE,)
    def eb(shape):
        return pl.BlockSpec(shape, lambda i: (i, 0))
    def wb(shape):
        return pl.BlockSpec(shape, lambda i: (0, 0))
    return pl.pallas_call(
        _edge_body,
        grid=grid,
        in_specs=[
            eb((BE, D)), eb((BE, D)), eb((BE, 16)),
            wb((D, 2 * D)), wb((D, 2 * D)), wb((1, 2 * D)), wb((1, 2 * D)),
            wb((2 * D, D)), wb((1, D)), wb((D, D)), wb((1, D)),
            wb((D, D)), wb((1, D)), wb((1, D)),
        ],
        out_specs=[eb((BE, D)), eb((BE, 1))],
        out_shape=[
            jax.ShapeDtypeStruct((E, D), jnp.float32),
            jax.ShapeDtypeStruct((E, 1), jnp.float32),
        ],
    )(hrow, hcol, relc, We1a, We1b, we1d, be1, We2, be2, We3, be3,
      Wc1, bc1, wc2t)


@functools.partial(
    pl.kernel,
    out_type=jax.ShapeDtypeStruct((NC, N, D), jnp.float32),
    mesh=_mesh,
    scratch_types=[
        pltpu.VMEM((8, 16), jnp.int32),
        pltpu.VMEM((CH2, D), jnp.float32),
        pltpu.VMEM_SHARED((N, D), jnp.float32),
        pltpu.SemaphoreType.DMA,
    ],
)
def _scatter_kernel(msg_hbm, row16_hbm, zh_hbm, agg_hbm,
                    idxr_v, msg_v, sh_agg, sem):
    cid = lax.axis_index("c")
    sid = lax.axis_index("s")
    wid = sid * NC + cid
    a0 = sid * AGG_STRIPE
    asz = jnp.where(sid == NS - 1, AGG_LAST, AGG_STRIPE)
    pltpu.sync_copy(zh_hbm.at[pl.ds(0, asz)], sh_agg.at[pl.ds(a0, asz)])
    plsc.subcore_barrier()

    cnt = CNT2_BASE + jnp.where(wid < CNT2_REM, 1, 0)

    def step(t, carry):
        c = wid + t * NW
        rr = c * 8
        e0 = c * CH2
        pltpu.sync_copy(row16_hbm.at[pl.ds(rr, 8)], idxr_v)
        pltpu.sync_copy(msg_hbm.at[pl.ds(e0, CH2)], msg_v)
        for j in range(8):
            pltpu.sync_copy(msg_v.at[pl.ds(j * 16, 16)],
                            sh_agg.at[idxr_v.at[j]], add=True)
        return carry

    lax.fori_loop(0, cnt, step, 0)
    plsc.subcore_barrier()
    pltpu.sync_copy(sh_agg.at[pl.ds(a0, asz)], agg_hbm.at[cid, pl.ds(a0, asz)])


@functools.partial(
    pl.kernel,
    out_type=jax.ShapeDtypeStruct((NC, CR, D), jnp.float32),
    mesh=_mesh,
    compiler_params=pltpu.CompilerParams(needs_layout_passes=False),
    scratch_types=[
        pltpu.VMEM((8, 16), jnp.int32),
        pltpu.VMEM((8, 16), jnp.int32),
        pltpu.VMEM((CH2, D), jnp.float32),
        pltpu.VMEM((CH2, 16), jnp.float32),
        pltpu.VMEM((CH2,), jnp.float32),
        pltpu.VMEM_SHARED((CR, D), jnp.float32),
        pltpu.SemaphoreType.DMA,
    ],
)
def _cscatter_kernel(cw_hbm, relc_hbm, row16_hbm, row8_hbm, zh_hbm, cacc_hbm,
                     idxr_v, idx8_v, cww_v, rel_v, cw_v, sh_cacc, sem):
    cid = lax.axis_index("c")
    sid = lax.axis_index("s")
    wid = sid * NC + cid
    c0 = sid * CACC_STRIPE
    pltpu.sync_copy(zh_hbm.at[pl.ds(0, CACC_STRIPE)], sh_cacc.at[pl.ds(c0, CACC_STRIPE)])
    pltpu.sync_copy(zh_hbm.at[pl.ds(0, CH2)], cww_v)
    plsc.subcore_barrier()

    cnt = CNT2_BASE + jnp.where(wid < CNT2_REM, 1, 0)
    lanes = lax.iota(jnp.int32, 16)
    zero16 = jnp.zeros((16,), jnp.float32)

    def step(t, carry):
        c = wid + t * NW
        rr = c * 8
        e0 = c * CH2
        pltpu.sync_copy(row16_hbm.at[pl.ds(rr, 8)], idxr_v)
        pltpu.sync_copy(row8_hbm.at[pl.ds(rr, 8)], idx8_v)
        pltpu.sync_copy(relc_hbm.at[pl.ds(e0, CH2)], rel_v)
        pltpu.sync_copy(cw_hbm.at[pl.ds(e0, CH2)], cw_v)
        for j in range(8):
            ev = lanes + j * 16
            r = idxr_v.at[j][...]
            slot = (r & 7) * 16
            cwv = cw_v[pl.ds(j * 16, 16)]
            for comp in range(3):
                rc = plsc.load_gather(rel_v, [ev, jnp.full((16,), comp, jnp.int32)])
                plsc.store_scatter(cww_v, [ev, slot + comp], cwv * rc)
        for j in range(8):
            pltpu.sync_copy(cww_v.at[pl.ds(j * 16, 16)],
                            sh_cacc.at[idx8_v.at[j]], add=True)
        for j in range(8):
            ev = lanes + j * 16
            slot = (idxr_v.at[j][...] & 7) * 16
            for comp in range(3):
                plsc.store_scatter(cww_v, [ev, slot + comp], zero16)
        return carry

    lax.fori_loop(0, cnt, step, 0)
    plsc.subcore_barrier()
    pltpu.sync_copy(sh_cacc.at[pl.ds(c0, CACC_STRIPE)],
                    cacc_hbm.at[cid, pl.ds(c0, CACC_STRIPE)])


BN = 1000


def _node_body(h, agg2, c16, cacc2, Wn1a, Wn1b, bn1, Wn2, bn2, Wn3, bn3,
               gamma, beta, hnew_o, c16_o):
    agg = agg2[0] + agg2[1]
    u = _silu(jnp.dot(h[...], Wn1a[...], preferred_element_type=jnp.float32)
              + jnp.dot(agg, Wn1b[...], preferred_element_type=jnp.float32)
              + bn1[...])
    u2 = _silu(jnp.dot(u, Wn2[...], preferred_element_type=jnp.float32) + bn2[...])
    x = h[...] + jnp.dot(u2, Wn3[...], preferred_element_type=jnp.float32) + bn3[...]
    mu = jnp.mean(x, axis=1, keepdims=True)
    xc = x - mu
    var = jnp.mean(xc * xc, axis=1, keepdims=True)
    hnew_o[...] = xc * lax.rsqrt(var + 1e-5) * gamma[...] + beta[...]
    c16_o[...] = c16[...] + cacc2[0] + cacc2[1]


def _node_call(h, agg2, c16, cacc2, Wn1a, Wn1b, bn1, Wn2, bn2, Wn3, bn3,
               gamma, beta):
    grid = (N // BN,)
    def nb(shape):
        return pl.BlockSpec(shape, lambda i: (i, 0))
    def pb(shape):
        return pl.BlockSpec(shape, lambda i: (0, i, 0))
    def wb(shape):
        return pl.BlockSpec(shape, lambda i: tuple(0 for _ in shape))
    return pl.pallas_call(
        _node_body,
        grid=grid,
        in_specs=[
            nb((BN, D)), pb((NC, BN, D)), nb((BN, 16)), pb((NC, BN, 16)),
            wb((D, 2 * D)), wb((D, 2 * D)), wb((1, 2 * D)),
            wb((2 * D, D)), wb((1, D)), wb((D, D)), wb((1, D)),
            wb((1, D)), wb((1, D)),
        ],
        out_specs=[nb((BN, D)), nb((BN, 16))],
        out_shape=[
            jax.ShapeDtypeStruct((N, D), jnp.float32),
            jax.ShapeDtypeStruct((N, 16), jnp.float32),
        ],
    )(h, agg2, c16, cacc2, Wn1a, Wn1b, bn1, Wn2, bn2, Wn3, bn3, gamma, beta)


def kernel(h, coords, edge_index, We1, be1, We2, be2, We3, be3,
           Wn1, bn1, Wn2, bn2, Wn3, bn3, Wc1, bc1, Wc2, gamma, beta):
    row = edge_index[0]
    col = edge_index[1]
    row2 = row.reshape(IDX_ROWS, GB)
    col2 = col.reshape(IDX_ROWS, GB)
    row16 = row.reshape(IDX2_ROWS, 16)
    col16 = col.reshape(IDX2_ROWS, 16)
    cx = coords[:, 0]
    cy = coords[:, 1]
    cz = coords[:, 2]
    c16 = jnp.zeros((N, 16), jnp.float32).at[:, :3].set(coords)

    hrow, hcol = _gather_kernel(h, row2, col2)
    relflat = _rel_kernel(cx, cy, cz, row16, col16)
    relc = relflat.reshape(E, 16)

    msg, cw = _edge_call(
        hrow, hcol, relc,
        We1[:D], We1[D:2 * D], We1[2 * D:], be1.reshape(1, 2 * D),
        We2, be2.reshape(1, D), We3, be3.reshape(1, D),
        Wc1, bc1.reshape(1, D), Wc2.reshape(1, D))

    zh = jnp.zeros((AGG_STRIPE, D), jnp.float32)
    row8 = (row >> 3).reshape(IDX2_ROWS, 16)
    agg2 = _scatter_kernel(msg, row16, zh)
    cacc2w = _cscatter_kernel(cw.reshape(E), relc, row16, row8, zh)
    cacc2 = cacc2w.reshape(NC, CR * 8, 16)[:, :N]

    h_new, c16_o = _node_call(
        h, agg2, c16, cacc2,
        Wn1[:D], Wn1[D:], bn1.reshape(1, 2 * D),
        Wn2, bn2.reshape(1, D), Wn3, bn3.reshape(1, D),
        gamma.reshape(1, D), beta.reshape(1, D))

    return h_new, c16_o[:, :3]
